# Initial kernel scaffold; baseline (speedup 1.0000x reference)
import jax
import jax.numpy as jnp
from jax.experimental import pallas as pl


def kernel(node_raw_features, retrieved_nodes, src_node_ids, dst_node_ids, node_interact_times, retrieved_indices, W_feat, b_feat, W_struct, b_struct, W_gcn, b_gcn, ln_g, ln_b, W_out, b_out):
    raise NotImplementedError("write your pallas kernel here")


# speedup vs baseline: 2.6367x; 2.6367x over previous
import functools

import jax
import jax.numpy as jnp
from jax import lax
from jax.experimental import pallas as pl
from jax.experimental.pallas import tpu as pltpu
from jax.experimental.pallas import tpu_sc as plsc

_NC = 2
_NS = 16
_NW = _NC * _NS

_D = 32
_F = 128
_MW = 16


def _sc_gather(raw, rn_pad, rindices, dst_ids, seq):
    b_total, k_demos = rindices.shape
    bpw = b_total // _NW
    seq_pad = (seq + 7) // 8 * 8
    n_chunks = seq // 16 + (1 if seq % 16 else 0)
    mesh = plsc.VectorSubcoreMesh(core_axis_name="c", subcore_axis_name="s",
                                  num_cores=_NC, num_subcores=_NS)

    @functools.partial(
        pl.kernel,
        out_type=[
            jax.ShapeDtypeStruct((b_total * seq, _F), jnp.float32),
            jax.ShapeDtypeStruct((b_total * seq, _MW), jnp.float32),
            jax.ShapeDtypeStruct((b_total, _F), jnp.float32),
        ],
        mesh=mesh,
        compiler_params=pltpu.CompilerParams(use_tc_tiling_on_sc=False),
        scratch_types=[
            pltpu.VMEM((bpw, k_demos), jnp.int32),
            pltpu.VMEM((bpw,), jnp.int32),
            pltpu.VMEM((k_demos, _F), jnp.int32),
            pltpu.VMEM((seq_pad, _F), jnp.float32),
            pltpu.VMEM((seq_pad, _F), jnp.float32),
            pltpu.VMEM((seq_pad, _F), jnp.float32),
            pltpu.VMEM((seq_pad, _F), jnp.float32),
            pltpu.VMEM((2 * seq, _F), jnp.float32),
            pltpu.VMEM((bpw * seq, _MW), jnp.float32),
            pltpu.VMEM((bpw, _F), jnp.float32),
            pltpu.SemaphoreType.DMA,
            pltpu.SemaphoreType.DMA,
            pltpu.SemaphoreType.DMA,
        ],
    )
    def sc_kernel(raw_hbm, rn_hbm, ri_hbm, dst_hbm, hs_hbm, mt_hbm, dstp_hbm,
                  myri, mydst, ids, f0, f1, f2, f3, hbuf, mbuf, dstp,
                  sem_a, sem_b, sem_c):
        wid = lax.axis_index("s") * _NC + lax.axis_index("c")
        base = wid * bpw
        pltpu.sync_copy(ri_hbm.at[pl.ds(base, bpw)], myri)
        pltpu.sync_copy(dst_hbm.at[pl.ds(base, bpw)], mydst)
        pltpu.async_copy(raw_hbm.at[mydst], dstp, sem_a).wait()
        pltpu.sync_copy(dstp, dstp_hbm.at[pl.ds(base, bpw)])

        lanes = lax.iota(jnp.int32, 16)
        lane0f = jnp.where(lanes == 0, 1.0, 0.0)
        lane1f = jnp.where(lanes == 1, 1.0, 0.0)
        fresh_i = jnp.where(lanes >= 16 * n_chunks - seq, 1, 0)
        fbufs = (f0, f1, f2, f3)

        def body(jp, _):
            for u in range(2):
                j = jp * 2 + u
                pltpu.async_copy(rn_hbm.at[myri.at[j]], ids, sem_a).wait()
                copies = [
                    pltpu.async_copy(
                        raw_hbm.at[ids.at[kk, pl.ds(0, seq_pad)]], fbufs[kk],
                        sem_b)
                    for kk in range(k_demos)
                ]
                curv = dstp[j, pl.ds(0, 16)].astype(jnp.int32)

                accv = jnp.zeros((16,), jnp.int32)
                for c in range(n_chunks):
                    off = 16 * c if c < n_chunks - 1 else seq - 16
                    vi = jnp.where(ids[0, pl.ds(off, 16)] > 0, 1, 0)
                    if off != 16 * c:
                        vi = vi * fresh_i
                    accv = accv + vi
                nv_s = accv[0]
                for i in range(1, 16):
                    nv_s = nv_s + accv[i]
                nv = jnp.full((16,), nv_s, jnp.int32)
                for cp in copies:
                    cp.wait()

                row0h = u * seq
                row0m = j * seq

                def lsum(l, _):
                    for h in range(0, _F, 16):
                        acc = fbufs[0][l, pl.ds(h, 16)]
                        for kk in range(1, k_demos):
                            acc = acc + fbufs[kk][l, pl.ds(h, 16)]
                        hbuf[row0h + l, pl.ds(h, 16)] = acc
                    cnt = jnp.zeros((16,), jnp.int32)
                    for kk in range(k_demos):
                        sk = fbufs[kk][l, pl.ds(0, 16)].astype(jnp.int32)
                        cnt = cnt + jnp.where(sk == curv, 1, 0)
                    lv = jnp.full((16,), l, jnp.int32)
                    wlf = (lane1f
                           * jnp.where(lv >= 1, 1.0, 0.0)
                           * jnp.where(lv < nv, 1.0, 0.0))
                    mbuf[row0m + l, :] = cnt.astype(jnp.float32) * lane0f + wlf
                    return 0

                lax.fori_loop(0, seq, lsum, 0)
            pltpu.sync_copy(
                hbuf, hs_hbm.at[pl.ds((base + jp * 2) * seq, 2 * seq)])
            return 0

        lax.fori_loop(0, bpw // 2, body, 0)
        pltpu.sync_copy(mbuf, mt_hbm.at[pl.ds(base * seq, bpw * seq)])

    return sc_kernel(raw, rn_pad, rindices, dst_ids)


def _post_body(hs_ref, mt_ref, dstp_ref, wfeat_ref, wstruct_ref, bfs_ref,
               wgcn_ref, bgcn_ref, lng_ref, lnb_ref, wout_ref, bout_ref,
               bfeat_ref, src_ref, dst_ref, *, seq, inv_k):
    hs = hs_ref[...]
    rows = hs.shape[0]
    nb = rows // seq
    mt = mt_ref[...]
    mc = mt[:, 0:1]
    w = mt[:, 1:2]
    proj = jnp.dot(hs, wfeat_ref[...],
                   preferred_element_type=jnp.float32) * inv_k
    fused = proj + bfs_ref[...] + (mc * inv_k) * wstruct_ref[...]
    mu = jnp.mean(fused, axis=-1, keepdims=True)
    var = jnp.mean((fused - mu) ** 2, axis=-1, keepdims=True)
    hist = (fused - mu) / jnp.sqrt(var + 1e-5) * lng_ref[...] + lnb_ref[...]
    xw = jnp.dot(hist, wgcn_ref[...], preferred_element_type=jnp.float32)
    deg = 1.0 + w
    dinv = 1.0 / jnp.sqrt(deg)
    dinv_m1 = jnp.concatenate(
        [jnp.ones((1, 1), jnp.float32), dinv[:-1, :]], axis=0)
    ec = w * dinv_m1 * dinv
    xw_m1 = jnp.concatenate(
        [jnp.zeros((1, _D), jnp.float32), xw[:-1, :]], axis=0)
    agg = (dinv * dinv) * xw + ec * xw_m1
    gnn = jnp.maximum(agg + bgcn_ref[...], 0.0)
    r = lax.broadcasted_iota(jnp.int32, (nb, rows), 1)
    g = lax.broadcasted_iota(jnp.int32, (nb, rows), 0) * seq
    sel = jnp.where(jnp.logical_and(r >= g, r < g + seq), 1.0, 0.0)
    pooled = jnp.dot(sel, gnn, preferred_element_type=jnp.float32) / float(seq)
    src_ref[...] = jnp.dot(pooled, wout_ref[...],
                           preferred_element_type=jnp.float32) + bout_ref[...]
    dstf = jnp.dot(dstp_ref[...], wfeat_ref[...],
                   preferred_element_type=jnp.float32) + bfeat_ref[...]
    dst_ref[...] = jnp.dot(dstf, wout_ref[...],
                           preferred_element_type=jnp.float32) + bout_ref[...]


def _post(hsum, meta, dstp, w_feat, w_struct, bfs, w_gcn, b_gcn, ln_g, ln_b,
          w_out, b_out, b_feat, seq, inv_k):
    b_total = dstp.shape[0]
    bb = 32
    grid = b_total // bb
    rows = bb * seq
    wspec = pl.BlockSpec((1, _D), lambda i: (0, 0))
    mspec = pl.BlockSpec((_D, _D), lambda i: (0, 0))
    return pl.pallas_call(
        functools.partial(_post_body, seq=seq, inv_k=inv_k),
        grid=(grid,),
        in_specs=[
            pl.BlockSpec((rows, _F), lambda i: (i, 0)),
            pl.BlockSpec((rows, _MW), lambda i: (i, 0)),
            pl.BlockSpec((bb, _F), lambda i: (i, 0)),
            pl.BlockSpec((_F, _D), lambda i: (0, 0)),
            wspec, wspec, mspec, wspec, wspec, wspec, mspec, wspec, wspec,
        ],
        out_specs=[
            pl.BlockSpec((bb, _D), lambda i: (i, 0)),
            pl.BlockSpec((bb, _D), lambda i: (i, 0)),
        ],
        out_shape=[
            jax.ShapeDtypeStruct((b_total, _D), jnp.float32),
            jax.ShapeDtypeStruct((b_total, _D), jnp.float32),
        ],
    )(hsum, meta, dstp, w_feat, w_struct, bfs, w_gcn, b_gcn, ln_g, ln_b,
      w_out, b_out, b_feat)


def kernel(node_raw_features, retrieved_nodes, src_node_ids, dst_node_ids,
           node_interact_times, retrieved_indices,
           W_feat, b_feat, W_struct, b_struct, W_gcn, b_gcn,
           ln_g, ln_b, W_out, b_out):
    seq = retrieved_nodes.shape[1]
    inv_k = 1.0 / retrieved_indices.shape[1]
    rn_pad = jnp.pad(retrieved_nodes, ((0, 0), (0, _F - seq)))
    hsum, meta, dstp = _sc_gather(node_raw_features, rn_pad,
                                  retrieved_indices, dst_node_ids, seq)
    bfs = (b_feat + b_struct).reshape(1, _D)
    return _post(hsum, meta, dstp, W_feat, W_struct.reshape(1, _D), bfs,
                 W_gcn, b_gcn.reshape(1, _D), ln_g.reshape(1, _D),
                 ln_b.reshape(1, _D), W_out, b_out.reshape(1, _D),
                 b_feat.reshape(1, _D), seq, inv_k)
